# baseline (speedup 1.0000x reference)
import jax
import jax.numpy as jnp
from jax import lax
from jax.experimental import pallas as pl
from jax.experimental.pallas import tpu as pltpu
from jax.experimental.pallas import tpu_sc as plsc

_QUES = 4096
_H = 2048
_3H = 3 * _H
_R = 512
_NB = _3H // _R

_NT = 32
_RPT = _3H // _NT
_G = 16
_NG = _RPT // _G
_CH = _H // 16


def _gi_kernel(sel_ref, q_ref, h_ref, ws_ref, bs_ref, bih_ref, wih_ref,
               pred_ref, gi_ref):
    del sel_ref
    k = pl.program_id(0)
    gi_ref[...] = jax.lax.dot_general(
        q_ref[...], wih_ref[...], (((1,), (1,)), ((), ())),
        preferred_element_type=jnp.float32) + bih_ref[...]

    @pl.when(k == 0)
    def _():
        val = (jnp.sum(q_ref[0, :] * ws_ref[0, :_QUES])
               + jnp.sum(h_ref[0, :] * ws_ref[0, _QUES:])
               + bs_ref[0, 0])
        pred_ref[...] = jnp.reshape(val, (1, 1))


def _gh_sc_kernel(whh_hbm, h_hbm, out_hbm, h_v, w_v, out_v):
    wid = lax.axis_index("s") * 2 + lax.axis_index("c")
    pltpu.sync_copy(h_hbm, h_v)
    lane = lax.iota(jnp.int32, 16)

    def group(g, carry):
        blk = wid * _NG + g
        pltpu.sync_copy(whh_hbm.at[blk], w_v)
        accs = [jnp.zeros((16,), jnp.float32) for _ in range(_G)]
        for c in range(_CH):
            hc = h_v[0, c * 16:(c + 1) * 16]
            for i in range(_G):
                accs[i] = accs[i] + w_v[i, c * 16:(c + 1) * 16] * hc
        vec = jnp.zeros((16,), jnp.float32)
        for i in range(_G):
            a = accs[i]
            s = a[0]
            for j in range(1, 16):
                s = s + a[j]
            vec = jnp.where(lane == i, s, vec)
        out_v[0, :] = vec
        pltpu.sync_copy(out_v, out_hbm.at[blk])
        return carry

    lax.fori_loop(0, _NG, group, 0)


def _combine_kernel(gi_ref, gh_ref, bhh_ref, h_ref, hout_ref):
    gi = gi_ref[...]
    gh = gh_ref[...] + bhh_ref[...]
    r = jax.nn.sigmoid(gi[:, :_H] + gh[:, :_H])
    z = jax.nn.sigmoid(gi[:, _H:2 * _H] + gh[:, _H:2 * _H])
    n = jnp.tanh(gi[:, 2 * _H:] + r * gh[:, 2 * _H:])
    hout_ref[...] = (1.0 - z) * n + z * h_ref[...]


def kernel(ques_h, score, initial_h, W_s, b_s, W_ih, W_hh, b_ih, b_hh):
    sel = (score < 0.5).astype(jnp.int32)
    q2 = ques_h.reshape(1, _QUES)
    h2 = initial_h.reshape(1, _H)
    bs2 = b_s.reshape(1, 1)
    bih2 = b_ih.reshape(1, _3H)
    bhh2 = b_hh.reshape(1, _3H)

    grid_spec = pltpu.PrefetchScalarGridSpec(
        num_scalar_prefetch=1,
        grid=(_NB,),
        in_specs=[
            pl.BlockSpec((1, _QUES), lambda k, sel: (0, 0)),
            pl.BlockSpec((1, _H), lambda k, sel: (0, 0)),
            pl.BlockSpec((1, _QUES + _H), lambda k, sel: (0, 0)),
            pl.BlockSpec((1, 1), lambda k, sel: (0, 0)),
            pl.BlockSpec((1, _R), lambda k, sel: (0, k)),
            pl.BlockSpec((_R, _QUES), lambda k, sel: (k, sel[0])),
        ],
        out_specs=[
            pl.BlockSpec((1, 1), lambda k, sel: (0, 0)),
            pl.BlockSpec((1, _R), lambda k, sel: (0, k)),
        ],
    )
    pred2, gi = pl.pallas_call(
        _gi_kernel,
        grid_spec=grid_spec,
        out_shape=[
            jax.ShapeDtypeStruct((1, 1), jnp.float32),
            jax.ShapeDtypeStruct((1, _3H), jnp.float32),
        ],
    )(sel, q2, h2, W_s, bs2, bih2, W_ih)

    whh3 = W_hh.reshape(_3H // _G, _G, _H)
    mesh = plsc.VectorSubcoreMesh(core_axis_name="c", subcore_axis_name="s")
    gh_part = pl.kernel(
        _gh_sc_kernel,
        mesh=mesh,
        out_type=jax.ShapeDtypeStruct((_3H // _G, 1, 16), jnp.float32),
        scratch_types=[
            pltpu.VMEM((1, _H), jnp.float32),
            pltpu.VMEM((_G, _H), jnp.float32),
            pltpu.VMEM((1, 16), jnp.float32),
        ],
    )(whh3, h2)
    gh = gh_part.reshape(1, _3H)

    hout = pl.pallas_call(
        _combine_kernel,
        out_shape=jax.ShapeDtypeStruct((1, _H), jnp.float32),
    )(gi, gh, bhh2, h2)

    return (pred2[0], ques_h, hout.reshape(1, 1, _H))


# speedup vs baseline: 2.0121x; 2.0121x over previous
import jax
import jax.numpy as jnp
from jax import lax
from jax.experimental import pallas as pl
from jax.experimental.pallas import tpu as pltpu
from jax.experimental.pallas import tpu_sc as plsc

_QUES = 4096
_H = 2048
_3H = 3 * _H
_R = 512
_NB = _3H // _R

_NT = 32
_RPT = _3H // _NT
_G = 16
_NG = _RPT // _G
_CH = _H // 16


def _gi_kernel(sel_ref, q_ref, h_ref, ws_ref, bs_ref, bih_ref, wih_ref,
               pred_ref, gi_ref):
    del sel_ref
    k = pl.program_id(0)
    gi_ref[...] = jax.lax.dot_general(
        q_ref[...], wih_ref[...], (((1,), (1,)), ((), ())),
        preferred_element_type=jnp.float32) + bih_ref[...]

    @pl.when(k == 0)
    def _():
        val = (jnp.sum(q_ref[0, :] * ws_ref[0, :_QUES])
               + jnp.sum(h_ref[0, :] * ws_ref[0, _QUES:])
               + bs_ref[0, 0])
        pred_ref[...] = jnp.reshape(val, (1, 1))


def _gh_sc_kernel(whh_hbm, h_hbm, out_hbm, h_v, w_v, out_v):
    wid = lax.axis_index("s") * 2 + lax.axis_index("c")
    pltpu.sync_copy(h_hbm, h_v)
    lane = lax.iota(jnp.int32, 16)

    def group(g, carry):
        blk = wid * _NG + g
        pltpu.sync_copy(whh_hbm.at[blk], w_v)
        vec = jnp.zeros((16,), jnp.float32)
        for half in range(2):
            def cbody(c, accs):
                hc = h_v[0, pl.ds(c * 16, 16)]
                return tuple(
                    accs[i] + w_v[half * 8 + i, pl.ds(c * 16, 16)] * hc
                    for i in range(8))

            zero8 = tuple(jnp.zeros((16,), jnp.float32) for _ in range(8))
            accs = lax.fori_loop(0, _CH, cbody, zero8, unroll=8)
            for i in range(8):
                a = accs[i]
                s = a[0]
                for j in range(1, 16):
                    s = s + a[j]
                vec = jnp.where(lane == half * 8 + i, s, vec)
        out_v[0, :] = vec
        pltpu.sync_copy(out_v, out_hbm.at[blk])
        return carry

    lax.fori_loop(0, _NG, group, 0)


def _combine_kernel(gi_ref, gh_ref, bhh_ref, h_ref, hout_ref):
    gi = gi_ref[...]
    gh = gh_ref[...] + bhh_ref[...]
    r = jax.nn.sigmoid(gi[:, :_H] + gh[:, :_H])
    z = jax.nn.sigmoid(gi[:, _H:2 * _H] + gh[:, _H:2 * _H])
    n = jnp.tanh(gi[:, 2 * _H:] + r * gh[:, 2 * _H:])
    hout_ref[...] = (1.0 - z) * n + z * h_ref[...]


def kernel(ques_h, score, initial_h, W_s, b_s, W_ih, W_hh, b_ih, b_hh):
    sel = (score < 0.5).astype(jnp.int32)
    q2 = ques_h.reshape(1, _QUES)
    h2 = initial_h.reshape(1, _H)
    bs2 = b_s.reshape(1, 1)
    bih2 = b_ih.reshape(1, _3H)
    bhh2 = b_hh.reshape(1, _3H)

    grid_spec = pltpu.PrefetchScalarGridSpec(
        num_scalar_prefetch=1,
        grid=(_NB,),
        in_specs=[
            pl.BlockSpec((1, _QUES), lambda k, sel: (0, 0)),
            pl.BlockSpec((1, _H), lambda k, sel: (0, 0)),
            pl.BlockSpec((1, _QUES + _H), lambda k, sel: (0, 0)),
            pl.BlockSpec((1, 1), lambda k, sel: (0, 0)),
            pl.BlockSpec((1, _R), lambda k, sel: (0, k)),
            pl.BlockSpec((_R, _QUES), lambda k, sel: (k, sel[0])),
        ],
        out_specs=[
            pl.BlockSpec((1, 1), lambda k, sel: (0, 0)),
            pl.BlockSpec((1, _R), lambda k, sel: (0, k)),
        ],
    )
    pred2, gi = pl.pallas_call(
        _gi_kernel,
        grid_spec=grid_spec,
        out_shape=[
            jax.ShapeDtypeStruct((1, 1), jnp.float32),
            jax.ShapeDtypeStruct((1, _3H), jnp.float32),
        ],
    )(sel, q2, h2, W_s, bs2, bih2, W_ih)

    whh3 = W_hh.reshape(_3H // _G, _G, _H)
    mesh = plsc.VectorSubcoreMesh(core_axis_name="c", subcore_axis_name="s")
    gh_part = pl.kernel(
        _gh_sc_kernel,
        mesh=mesh,
        out_type=jax.ShapeDtypeStruct((_3H // _G, 1, 16), jnp.float32),
        scratch_types=[
            pltpu.VMEM((1, _H), jnp.float32),
            pltpu.VMEM((_G, _H), jnp.float32),
            pltpu.VMEM((1, 16), jnp.float32),
        ],
    )(whh3, h2)
    gh = gh_part.reshape(1, _3H)

    hout = pl.pallas_call(
        _combine_kernel,
        out_shape=jax.ShapeDtypeStruct((1, _H), jnp.float32),
    )(gi, gh, bhh2, h2)

    return (pred2[0], ques_h, hout.reshape(1, 1, _H))


# speedup vs baseline: 2.3177x; 1.1519x over previous
import jax
import jax.numpy as jnp
from jax import lax
from jax.experimental import pallas as pl
from jax.experimental.pallas import tpu as pltpu
from jax.experimental.pallas import tpu_sc as plsc

_QUES = 4096
_H = 2048
_3H = 3 * _H
_R = 512
_NB = _3H // _R

_NT = 32
_RPT = _3H // _NT
_G = 16
_NG = _RPT // _G
_CH = _H // 16


def _gi_kernel(sel_ref, q_ref, h_ref, ws_ref, bs_ref, bih_ref, wih_ref,
               pred_ref, gi_ref):
    del sel_ref
    k = pl.program_id(0)
    gi_ref[...] = jax.lax.dot_general(
        q_ref[...], wih_ref[...], (((1,), (1,)), ((), ())),
        preferred_element_type=jnp.float32) + bih_ref[...]

    @pl.when(k == 0)
    def _():
        val = (jnp.sum(q_ref[0, :] * ws_ref[0, :_QUES])
               + jnp.sum(h_ref[0, :] * ws_ref[0, _QUES:])
               + bs_ref[0, 0])
        pred_ref[...] = jnp.reshape(val, (1, 1))


def _gh_sc_kernel(whh_hbm, h_hbm, out_hbm, h_v, w0_v, w1_v, out_v,
                  sem0, sem1):
    wid = lax.axis_index("s") * 2 + lax.axis_index("c")
    base = wid * _NG
    pltpu.sync_copy(h_hbm, h_v)
    lane = lax.iota(jnp.int32, 16)

    def compute_group(w_v, g):
        vec = jnp.zeros((16,), jnp.float32)
        for half in range(2):
            def cbody(c, accs):
                hc = h_v[0, pl.ds(c * 16, 16)]
                return tuple(
                    accs[i] + w_v[half * 8 + i, pl.ds(c * 16, 16)] * hc
                    for i in range(8))

            zero8 = tuple(jnp.zeros((16,), jnp.float32) for _ in range(8))
            accs = lax.fori_loop(0, _CH, cbody, zero8, unroll=8)
            for i in range(8):
                a = accs[i]
                s = a[0]
                for j in range(1, 16):
                    s = s + a[j]
                vec = jnp.where(lane == half * 8 + i, s, vec)
        out_v[0, pl.ds(g * 16, 16)] = vec

    pltpu.make_async_copy(whh_hbm.at[base], w0_v, sem0).start()

    def pair(p, carry):
        g0 = 2 * p
        pltpu.make_async_copy(whh_hbm.at[base + g0 + 1], w1_v, sem1).start()
        pltpu.make_async_copy(whh_hbm.at[base + g0], w0_v, sem0).wait()
        compute_group(w0_v, g0)

        @pl.when(p < _NG // 2 - 1)
        def _():
            pltpu.make_async_copy(whh_hbm.at[base + g0 + 2], w0_v, sem0).start()

        pltpu.make_async_copy(whh_hbm.at[base + g0 + 1], w1_v, sem1).wait()
        compute_group(w1_v, g0 + 1)
        return carry

    lax.fori_loop(0, _NG // 2, pair, 0)
    pltpu.sync_copy(out_v, out_hbm.at[wid])


def _combine_kernel(gi_ref, gh_ref, bhh_ref, h_ref, hout_ref):
    gi = gi_ref[...]
    gh = gh_ref[...] + bhh_ref[...]
    r = jax.nn.sigmoid(gi[:, :_H] + gh[:, :_H])
    z = jax.nn.sigmoid(gi[:, _H:2 * _H] + gh[:, _H:2 * _H])
    n = jnp.tanh(gi[:, 2 * _H:] + r * gh[:, 2 * _H:])
    hout_ref[...] = (1.0 - z) * n + z * h_ref[...]


def kernel(ques_h, score, initial_h, W_s, b_s, W_ih, W_hh, b_ih, b_hh):
    sel = (score < 0.5).astype(jnp.int32)
    q2 = ques_h.reshape(1, _QUES)
    h2 = initial_h.reshape(1, _H)
    bs2 = b_s.reshape(1, 1)
    bih2 = b_ih.reshape(1, _3H)
    bhh2 = b_hh.reshape(1, _3H)

    whh3 = W_hh.reshape(_3H // _G, _G, _H)
    mesh = plsc.VectorSubcoreMesh(core_axis_name="c", subcore_axis_name="s")
    gh_part = pl.kernel(
        _gh_sc_kernel,
        mesh=mesh,
        out_type=jax.ShapeDtypeStruct((_NT, 1, _RPT), jnp.float32),
        scratch_types=[
            pltpu.VMEM((1, _H), jnp.float32),
            pltpu.VMEM((_G, _H), jnp.float32),
            pltpu.VMEM((_G, _H), jnp.float32),
            pltpu.VMEM((1, _RPT), jnp.float32),
            pltpu.SemaphoreType.DMA,
            pltpu.SemaphoreType.DMA,
        ],
    )(whh3, h2)
    gh = gh_part.reshape(1, _3H)

    grid_spec = pltpu.PrefetchScalarGridSpec(
        num_scalar_prefetch=1,
        grid=(_NB,),
        in_specs=[
            pl.BlockSpec((1, _QUES), lambda k, sel: (0, 0)),
            pl.BlockSpec((1, _H), lambda k, sel: (0, 0)),
            pl.BlockSpec((1, _QUES + _H), lambda k, sel: (0, 0)),
            pl.BlockSpec((1, 1), lambda k, sel: (0, 0)),
            pl.BlockSpec((1, _R), lambda k, sel: (0, k)),
            pl.BlockSpec((_R, _QUES), lambda k, sel: (k, sel[0])),
        ],
        out_specs=[
            pl.BlockSpec((1, 1), lambda k, sel: (0, 0)),
            pl.BlockSpec((1, _R), lambda k, sel: (0, k)),
        ],
    )
    pred2, gi = pl.pallas_call(
        _gi_kernel,
        grid_spec=grid_spec,
        out_shape=[
            jax.ShapeDtypeStruct((1, 1), jnp.float32),
            jax.ShapeDtypeStruct((1, _3H), jnp.float32),
        ],
    )(sel, q2, h2, W_s, bs2, bih2, W_ih)

    hout = pl.pallas_call(
        _combine_kernel,
        out_shape=jax.ShapeDtypeStruct((1, _H), jnp.float32),
    )(gi, gh, bhh2, h2)

    return (pred2[0], ques_h, hout.reshape(1, 1, _H))


# speedup vs baseline: 3.0084x; 1.2980x over previous
import jax
import jax.numpy as jnp
from jax.experimental import pallas as pl
from jax.experimental.pallas import tpu as pltpu

_QUES = 4096
_H = 2048
_R = 256
_JB = _H // _R


def _sp_kernel(sel_ref, q_ref, h_ref, ws_ref, bs_ref, bih_ref, bhh_ref,
               wih_ref, whh_ref, pred_ref, hout_ref, r_scr, z_scr):
    del sel_ref
    gate = pl.program_id(0)
    j = pl.program_id(1)

    gi = jax.lax.dot_general(
        q_ref[...], wih_ref[...], (((1,), (1,)), ((), ())),
        preferred_element_type=jnp.float32) + bih_ref[...]
    gh = jax.lax.dot_general(
        h_ref[...], whh_ref[...], (((1,), (1,)), ((), ())),
        preferred_element_type=jnp.float32) + bhh_ref[...]

    @pl.when(jnp.logical_and(gate == 0, j == 0))
    def _():
        val = (jnp.sum(q_ref[0, :] * ws_ref[0, :_QUES])
               + jnp.sum(h_ref[0, :] * ws_ref[0, _QUES:])
               + bs_ref[0, 0])
        pred_ref[...] = jnp.reshape(val, (1, 1))

    @pl.when(gate == 0)
    def _():
        r_scr[:, pl.ds(j * _R, _R)] = jax.nn.sigmoid(gi + gh)

    @pl.when(gate == 1)
    def _():
        z_scr[:, pl.ds(j * _R, _R)] = jax.nn.sigmoid(gi + gh)

    @pl.when(gate == 2)
    def _():
        r = r_scr[:, pl.ds(j * _R, _R)]
        z = z_scr[:, pl.ds(j * _R, _R)]
        n = jnp.tanh(gi + r * gh)
        h_blk = h_ref[:, pl.ds(j * _R, _R)]
        hout_ref[...] = (1.0 - z) * n + z * h_blk


def kernel(ques_h, score, initial_h, W_s, b_s, W_ih, W_hh, b_ih, b_hh):
    sel = (score < 0.5).astype(jnp.int32)
    q2 = ques_h.reshape(1, _QUES)
    h2 = initial_h.reshape(1, _H)
    bs2 = b_s.reshape(1, 1)
    bih2 = b_ih.reshape(1, 3 * _H)
    bhh2 = b_hh.reshape(1, 3 * _H)

    grid_spec = pltpu.PrefetchScalarGridSpec(
        num_scalar_prefetch=1,
        grid=(3, _JB),
        in_specs=[
            pl.BlockSpec((1, _QUES), lambda i, j, sel: (0, 0)),
            pl.BlockSpec((1, _H), lambda i, j, sel: (0, 0)),
            pl.BlockSpec((1, _QUES + _H), lambda i, j, sel: (0, 0)),
            pl.BlockSpec((1, 1), lambda i, j, sel: (0, 0)),
            pl.BlockSpec((1, _R), lambda i, j, sel: (0, i * _JB + j)),
            pl.BlockSpec((1, _R), lambda i, j, sel: (0, i * _JB + j)),
            pl.BlockSpec((_R, _QUES), lambda i, j, sel: (i * _JB + j, sel[0])),
            pl.BlockSpec((_R, _H), lambda i, j, sel: (i * _JB + j, 0)),
        ],
        out_specs=[
            pl.BlockSpec((1, 1), lambda i, j, sel: (0, 0)),
            pl.BlockSpec((1, _R), lambda i, j, sel: (0, j)),
        ],
        scratch_shapes=[
            pltpu.VMEM((1, _H), jnp.float32),
            pltpu.VMEM((1, _H), jnp.float32),
        ],
    )

    pred2, hout = pl.pallas_call(
        _sp_kernel,
        grid_spec=grid_spec,
        out_shape=[
            jax.ShapeDtypeStruct((1, 1), jnp.float32),
            jax.ShapeDtypeStruct((1, _H), jnp.float32),
        ],
    )(sel, q2, h2, W_s, bs2, bih2, bhh2, W_ih, W_hh)

    return (pred2[0], ques_h, hout.reshape(1, 1, _H))
